# Initial kernel scaffold; baseline (speedup 1.0000x reference)
import jax
import jax.numpy as jnp
from jax.experimental import pallas as pl


def kernel(graph_edge_index, diff_edge_index, heat, edge_weight, W0, b0, a0, gamma0, beta0, pa0, W1, b1, a1, gamma1, beta1, pa1):
    raise NotImplementedError("write your pallas kernel here")


# speedup vs baseline: 1.3639x; 1.3639x over previous
import functools

import jax
import jax.numpy as jnp
from jax import lax
from jax.experimental import pallas as pl
from jax.experimental.pallas import tpu as pltpu
from jax.experimental.pallas import tpu_sc as plsc

_NC = 2
_NS = 16
_NW = _NC * _NS
_L = 16

_EPS = 1e-5


def _mm_in_body(wt_ref, x_ref, o_ref):
    o_ref[...] = lax.dot_general(
        wt_ref[...], x_ref[...], (((1,), (1,)), ((), ())),
        preferred_element_type=jnp.float32, precision=lax.Precision.HIGHEST)


def _bn_chain(agg, b, a, g, be, pa):
    z = agg + b
    z = jnp.where(z > 0, z, a * z)
    n = z.shape[1]
    mu = jnp.sum(z, axis=1, keepdims=True) / n
    zc = z - mu
    var = jnp.sum(zc * zc, axis=1, keepdims=True) / n
    zn = g * zc * lax.rsqrt(var + _EPS) + be
    return jnp.where(zn > 0, zn, pa * zn)


def _mid_body(agg_ref, b_ref, a_ref, g_ref, be_ref, pa_ref, w_ref, o_ref):
    i = pl.program_id(0)
    zp = _bn_chain(agg_ref[...], b_ref[...], a_ref[0, 0], g_ref[...],
                   be_ref[...], pa_ref[0, 0])
    contrib = lax.dot_general(
        w_ref[...], zp, (((0,), (0,)), ((), ())),
        preferred_element_type=jnp.float32, precision=lax.Precision.HIGHEST)

    @pl.when(i == 0)
    def _():
        o_ref[...] = contrib

    @pl.when(i > 0)
    def _():
        o_ref[...] += contrib


def _final_body(agg_ref, b_ref, a_ref, g_ref, be_ref, pa_ref, o_ref):
    zp = _bn_chain(agg_ref[...], b_ref[...], a_ref[0, 0], g_ref[...],
                   be_ref[...], pa_ref[0, 0])
    o_ref[...] = zp.T


def _mm_in(w_t, x):
    dout, din = w_t.shape
    n = x.shape[0]
    blk = 64
    return pl.pallas_call(
        _mm_in_body,
        grid=(dout // blk,),
        in_specs=[
            pl.BlockSpec((blk, din), lambda i: (i, 0)),
            pl.BlockSpec((n, din), lambda i: (0, 0)),
        ],
        out_specs=pl.BlockSpec((blk, n), lambda i: (i, 0)),
        out_shape=jax.ShapeDtypeStruct((dout, n), jnp.float32),
    )(w_t, x)


def _mid(agg, b, a, g, be, pa, w):
    d, n = agg.shape
    dout = w.shape[1]
    blk = 64
    col = lambda i: (i, 0)
    scal = pl.BlockSpec((1, 1), lambda i: (0, 0), memory_space=pltpu.SMEM)
    return pl.pallas_call(
        _mid_body,
        grid=(d // blk,),
        in_specs=[
            pl.BlockSpec((blk, n), col),
            pl.BlockSpec((blk, 1), col),
            scal,
            pl.BlockSpec((blk, 1), col),
            pl.BlockSpec((blk, 1), col),
            scal,
            pl.BlockSpec((blk, dout), col),
        ],
        out_specs=pl.BlockSpec((dout, n), lambda i: (0, 0)),
        out_shape=jax.ShapeDtypeStruct((dout, n), jnp.float32),
    )(agg, b, a, g, be, pa, w)


def _final(agg, b, a, g, be, pa):
    d, n = agg.shape
    blk = 128
    col = lambda i: (i, 0)
    scal = pl.BlockSpec((1, 1), lambda i: (0, 0), memory_space=pltpu.SMEM)
    return pl.pallas_call(
        _final_body,
        grid=(d // blk,),
        in_specs=[
            pl.BlockSpec((blk, n), col),
            pl.BlockSpec((blk, 1), col),
            scal,
            pl.BlockSpec((blk, 1), col),
            pl.BlockSpec((blk, 1), col),
            scal,
        ],
        out_specs=pl.BlockSpec((n, blk), lambda i: (0, i)),
        out_shape=jax.ShapeDtypeStruct((n, d), jnp.float32),
    )(agg, b, a, g, be, pa)


def _sc_scatter(y_t, src, dst, ew):
    d, n = y_t.shape
    e = src.shape[0]
    cols_per_tile = d // _NW
    pass_cols = 4
    npass = cols_per_tile // pass_cols
    chunk = 4000
    nchunk = e // chunk
    groups = chunk // _L

    mesh = plsc.VectorSubcoreMesh(core_axis_name="c", subcore_axis_name="s")

    @functools.partial(
        pl.kernel,
        out_type=jax.ShapeDtypeStruct((d, n), jnp.float32),
        mesh=mesh,
        compiler_params=pltpu.CompilerParams(needs_layout_passes=False),
        scratch_types=[
            pltpu.VMEM((pass_cols, n), jnp.float32),
            pltpu.VMEM((pass_cols, n), jnp.float32),
            pltpu.VMEM((chunk,), jnp.int32),
            pltpu.VMEM((chunk,), jnp.int32),
            pltpu.VMEM((chunk,), jnp.float32),
        ],
    )
    def k(y_hbm, src_hbm, dst_hbm, ew_hbm, out_hbm, xrows, acc, sbuf, dbuf,
          wbuf):
        cid = lax.axis_index("c")
        sid = lax.axis_index("s")
        wid = sid * _NC + cid
        for p in range(npass):
            base = wid * cols_per_tile + p * pass_cols
            pltpu.sync_copy(y_hbm.at[pl.ds(base, pass_cols)], xrows)

            def zero_body(i, carry):
                zv = jnp.zeros((_L,), jnp.float32)
                for c in range(pass_cols):
                    acc[c, pl.ds(i * _L, _L)] = zv
                return carry

            lax.fori_loop(0, n // _L, zero_body, 0)

            def chunk_body(ch, carry):
                off = pl.multiple_of(ch * chunk, 8)
                pltpu.sync_copy(src_hbm.at[pl.ds(off, chunk)], sbuf)
                pltpu.sync_copy(dst_hbm.at[pl.ds(off, chunk)], dbuf)
                pltpu.sync_copy(ew_hbm.at[pl.ds(off, chunk)], wbuf)

                def g_body(gi, c2):
                    s_idx = sbuf[pl.ds(gi * _L, _L)]
                    d_idx = dbuf[pl.ds(gi * _L, _L)]
                    w = wbuf[pl.ds(gi * _L, _L)]
                    for c in range(pass_cols):
                        cvec = jnp.full((_L,), c, jnp.int32)
                        v = plsc.load_gather(xrows, [cvec, s_idx])
                        plsc.addupdate_scatter(acc, [cvec, d_idx], v * w)
                    return c2

                lax.fori_loop(0, groups, g_body, 0)
                return carry

            lax.fori_loop(0, nchunk, chunk_body, 0)
            pltpu.sync_copy(acc, out_hbm.at[pl.ds(base, pass_cols)])

    return k(y_t, src, dst, ew)


def kernel(graph_edge_index, diff_edge_index, heat, edge_weight,
           W0, b0, a0, gamma0, beta0, pa0,
           W1, b1, a1, gamma1, beta1, pa1):
    src = diff_edge_index[0]
    dst = diff_edge_index[1]
    d = W0.shape[1]

    b0c = b0.reshape(d, 1)
    g0c = gamma0.reshape(d, 1)
    be0c = beta0.reshape(d, 1)
    b1c = b1.reshape(d, 1)
    g1c = gamma1.reshape(d, 1)
    be1c = beta1.reshape(d, 1)
    a0s = a0.reshape(1, 1)
    pa0s = pa0.reshape(1, 1)
    a1s = a1.reshape(1, 1)
    pa1s = pa1.reshape(1, 1)

    y0 = _mm_in(W0.T, heat)
    agg0 = _sc_scatter(y0, src, dst, edge_weight)
    y1 = _mid(agg0, b0c, a0s, g0c, be0c, pa0s, W1)
    agg1 = _sc_scatter(y1, src, dst, edge_weight)
    return _final(agg1, b1c, a1s, g1c, be1c, pa1s)


# speedup vs baseline: 1.6966x; 1.2440x over previous
import functools

import jax
import jax.numpy as jnp
from jax import lax
from jax.experimental import pallas as pl
from jax.experimental.pallas import tpu as pltpu
from jax.experimental.pallas import tpu_sc as plsc

_NC = 2
_NS = 16
_NW = _NC * _NS
_L = 16

_EPS = 1e-5


def _mm_in_body(wt_ref, x_ref, o_ref):
    o_ref[...] = lax.dot_general(
        wt_ref[...], x_ref[...], (((1,), (1,)), ((), ())),
        preferred_element_type=jnp.float32, precision=lax.Precision.HIGHEST)


def _bn_chain(agg, b, a, g, be, pa):
    z = agg + b
    z = jnp.where(z > 0, z, a * z)
    n = z.shape[1]
    mu = jnp.sum(z, axis=1, keepdims=True) / n
    zc = z - mu
    var = jnp.sum(zc * zc, axis=1, keepdims=True) / n
    zn = g * zc * lax.rsqrt(var + _EPS) + be
    return jnp.where(zn > 0, zn, pa * zn)


def _mid_body(agg_ref, b_ref, a_ref, g_ref, be_ref, pa_ref, w_ref, o_ref):
    i = pl.program_id(0)
    zp = _bn_chain(agg_ref[...], b_ref[...], a_ref[0, 0], g_ref[...],
                   be_ref[...], pa_ref[0, 0])
    contrib = lax.dot_general(
        w_ref[...], zp, (((0,), (0,)), ((), ())),
        preferred_element_type=jnp.float32, precision=lax.Precision.HIGHEST)

    @pl.when(i == 0)
    def _():
        o_ref[...] = contrib

    @pl.when(i > 0)
    def _():
        o_ref[...] += contrib


def _final_body(agg_ref, b_ref, a_ref, g_ref, be_ref, pa_ref, o_ref):
    zp = _bn_chain(agg_ref[...], b_ref[...], a_ref[0, 0], g_ref[...],
                   be_ref[...], pa_ref[0, 0])
    o_ref[...] = zp.T


def _mm_in(w_t, x):
    dout, din = w_t.shape
    n = x.shape[0]
    blk = 64
    return pl.pallas_call(
        _mm_in_body,
        grid=(dout // blk,),
        in_specs=[
            pl.BlockSpec((blk, din), lambda i: (i, 0)),
            pl.BlockSpec((n, din), lambda i: (0, 0)),
        ],
        out_specs=pl.BlockSpec((blk, n), lambda i: (i, 0)),
        out_shape=jax.ShapeDtypeStruct((dout, n), jnp.float32),
    )(w_t, x)


def _mid(agg, b, a, g, be, pa, w):
    d, n = agg.shape
    dout = w.shape[1]
    blk = 64
    col = lambda i: (i, 0)
    scal = pl.BlockSpec((1, 1), lambda i: (0, 0), memory_space=pltpu.SMEM)
    return pl.pallas_call(
        _mid_body,
        grid=(d // blk,),
        in_specs=[
            pl.BlockSpec((blk, n), col),
            pl.BlockSpec((blk, 1), col),
            scal,
            pl.BlockSpec((blk, 1), col),
            pl.BlockSpec((blk, 1), col),
            scal,
            pl.BlockSpec((blk, dout), col),
        ],
        out_specs=pl.BlockSpec((dout, n), lambda i: (0, 0)),
        out_shape=jax.ShapeDtypeStruct((dout, n), jnp.float32),
    )(agg, b, a, g, be, pa, w)


def _final(agg, b, a, g, be, pa):
    d, n = agg.shape
    blk = 128
    col = lambda i: (i, 0)
    scal = pl.BlockSpec((1, 1), lambda i: (0, 0), memory_space=pltpu.SMEM)
    return pl.pallas_call(
        _final_body,
        grid=(d // blk,),
        in_specs=[
            pl.BlockSpec((blk, n), col),
            pl.BlockSpec((blk, 1), col),
            scal,
            pl.BlockSpec((blk, 1), col),
            pl.BlockSpec((blk, 1), col),
            scal,
        ],
        out_specs=pl.BlockSpec((n, blk), lambda i: (0, i)),
        out_shape=jax.ShapeDtypeStruct((n, d), jnp.float32),
    )(agg, b, a, g, be, pa)


def _sc_scatter(y_t, edata):
    d, n = y_t.shape
    e = edata.shape[1]
    cols_per_tile = d // _NW
    pass_cols = 4
    npass = cols_per_tile // pass_cols
    chunk = 3200
    nchunk = e // chunk
    groups = chunk // _L

    mesh = plsc.VectorSubcoreMesh(core_axis_name="c", subcore_axis_name="s")

    @functools.partial(
        pl.kernel,
        out_type=jax.ShapeDtypeStruct((d, n), jnp.float32),
        mesh=mesh,
        compiler_params=pltpu.CompilerParams(needs_layout_passes=False,
                                             disable_bounds_checks=True),
        scratch_types=[
            pltpu.VMEM((pass_cols, n), jnp.float32),
            pltpu.VMEM((pass_cols, n), jnp.float32),
            pltpu.VMEM((2, 3, chunk), jnp.int32),
            pltpu.SemaphoreType.DMA,
            pltpu.SemaphoreType.DMA,
        ],
    )
    def k(y_hbm, ed_hbm, out_hbm, xrows, acc, ebuf, sem0, sem1):
        cid = lax.axis_index("c")
        sid = lax.axis_index("s")
        wid = sid * _NC + cid
        sems = (sem0, sem1)

        def start(ch, p):
            off = pl.multiple_of(ch * chunk, 8)
            pltpu.async_copy(ed_hbm.at[:, pl.ds(off, chunk)], ebuf.at[p],
                             sems[p])

        def drain(p):
            pltpu.make_async_copy(ed_hbm.at[:, pl.ds(0, chunk)], ebuf.at[p],
                                  sems[p]).wait()

        def compute(p):
            def g_body(gi, c2):
                s_idx = ebuf[p, 0, pl.ds(gi * _L, _L)]
                d_idx = ebuf[p, 1, pl.ds(gi * _L, _L)]
                w = plsc.bitcast(ebuf[p, 2, pl.ds(gi * _L, _L)], jnp.float32)
                for c in range(pass_cols):
                    cvec = jnp.full((_L,), c, jnp.int32)
                    v = plsc.load_gather(xrows, [cvec, s_idx])
                    plsc.addupdate_scatter(acc, [cvec, d_idx], v * w)
                return c2

            lax.fori_loop(0, groups, g_body, 0, unroll=5)

        for p in range(npass):
            base = wid * cols_per_tile + p * pass_cols
            pltpu.sync_copy(y_hbm.at[pl.ds(base, pass_cols)], xrows)

            def zero_body(i, carry):
                zv = jnp.zeros((_L,), jnp.float32)
                for c in range(pass_cols):
                    acc[c, pl.ds(i * _L, _L)] = zv
                return carry

            lax.fori_loop(0, n // _L, zero_body, 0, unroll=5)

            start(0, 0)

            def pair_body(i2, carry):
                ch = i2 * 2
                start(ch + 1, 1)
                drain(0)
                compute(0)

                @pl.when(ch + 2 < nchunk)
                def _():
                    start(ch + 2, 0)

                drain(1)
                compute(1)
                return carry

            lax.fori_loop(0, nchunk // 2, pair_body, 0)
            pltpu.sync_copy(acc, out_hbm.at[pl.ds(base, pass_cols)])

    return k(y_t, edata)


def kernel(graph_edge_index, diff_edge_index, heat, edge_weight,
           W0, b0, a0, gamma0, beta0, pa0,
           W1, b1, a1, gamma1, beta1, pa1):
    ew_i32 = lax.bitcast_convert_type(edge_weight, jnp.int32)
    edata = jnp.concatenate(
        [diff_edge_index, ew_i32[None, :]], axis=0)
    d = W0.shape[1]

    b0c = b0.reshape(d, 1)
    g0c = gamma0.reshape(d, 1)
    be0c = beta0.reshape(d, 1)
    b1c = b1.reshape(d, 1)
    g1c = gamma1.reshape(d, 1)
    be1c = beta1.reshape(d, 1)
    a0s = a0.reshape(1, 1)
    pa0s = pa0.reshape(1, 1)
    a1s = a1.reshape(1, 1)
    pa1s = pa1.reshape(1, 1)

    y0 = _mm_in(W0.T, heat)
    agg0 = _sc_scatter(y0, edata)
    y1 = _mid(agg0, b0c, a0s, g0c, be0c, pa0s, W1)
    agg1 = _sc_scatter(y1, edata)
    return _final(agg1, b1c, a1s, g1c, be1c, pa1s)


# speedup vs baseline: 4.0042x; 2.3601x over previous
import functools

import jax
import jax.numpy as jnp
from jax import lax
from jax.experimental import pallas as pl
from jax.experimental.pallas import tpu as pltpu
from jax.experimental.pallas import tpu_sc as plsc

_NC = 2
_NS = 16
_NW = _NC * _NS
_L = 16

_EPS = 1e-5


def _mm_in_body(wt_ref, x_ref, o_ref):
    o_ref[...] = lax.dot_general(
        wt_ref[...], x_ref[...], (((1,), (1,)), ((), ())),
        preferred_element_type=jnp.float32, precision=lax.Precision.HIGHEST)


def _bn_chain(agg, b, a, g, be, pa):
    z = agg + b
    z = jnp.where(z > 0, z, a * z)
    n = z.shape[1]
    mu = jnp.sum(z, axis=1, keepdims=True) / n
    zc = z - mu
    var = jnp.sum(zc * zc, axis=1, keepdims=True) / n
    zn = g * zc * lax.rsqrt(var + _EPS) + be
    return jnp.where(zn > 0, zn, pa * zn)


def _mid_body(agg_ref, b_ref, a_ref, g_ref, be_ref, pa_ref, w_ref, o_ref):
    i = pl.program_id(0)
    zp = _bn_chain(agg_ref[...], b_ref[...], a_ref[0, 0], g_ref[...],
                   be_ref[...], pa_ref[0, 0])
    contrib = lax.dot_general(
        w_ref[...], zp, (((0,), (0,)), ((), ())),
        preferred_element_type=jnp.float32, precision=lax.Precision.HIGHEST)

    @pl.when(i == 0)
    def _():
        o_ref[...] = contrib

    @pl.when(i > 0)
    def _():
        o_ref[...] += contrib


def _final_body(agg_ref, b_ref, a_ref, g_ref, be_ref, pa_ref, o_ref):
    zp = _bn_chain(agg_ref[...], b_ref[...], a_ref[0, 0], g_ref[...],
                   be_ref[...], pa_ref[0, 0])
    o_ref[...] = zp.T


def _mm_in(w_t, x):
    dout, din = w_t.shape
    n = x.shape[0]
    blk = 64
    return pl.pallas_call(
        _mm_in_body,
        grid=(dout // blk,),
        in_specs=[
            pl.BlockSpec((blk, din), lambda i: (i, 0)),
            pl.BlockSpec((n, din), lambda i: (0, 0)),
        ],
        out_specs=pl.BlockSpec((blk, n), lambda i: (i, 0)),
        out_shape=jax.ShapeDtypeStruct((dout, n), jnp.float32),
    )(w_t, x)


def _mid(agg, b, a, g, be, pa, w):
    d, n = agg.shape
    dout = w.shape[1]
    blk = 64
    col = lambda i: (i, 0)
    scal = pl.BlockSpec((1, 1), lambda i: (0, 0), memory_space=pltpu.SMEM)
    return pl.pallas_call(
        _mid_body,
        grid=(d // blk,),
        in_specs=[
            pl.BlockSpec((blk, n), col),
            pl.BlockSpec((blk, 1), col),
            scal,
            pl.BlockSpec((blk, 1), col),
            pl.BlockSpec((blk, 1), col),
            scal,
            pl.BlockSpec((blk, dout), col),
        ],
        out_specs=pl.BlockSpec((dout, n), lambda i: (0, 0)),
        out_shape=jax.ShapeDtypeStruct((dout, n), jnp.float32),
    )(agg, b, a, g, be, pa, w)


def _final(agg, b, a, g, be, pa):
    d, n = agg.shape
    blk = 128
    col = lambda i: (i, 0)
    scal = pl.BlockSpec((1, 1), lambda i: (0, 0), memory_space=pltpu.SMEM)
    return pl.pallas_call(
        _final_body,
        grid=(d // blk,),
        in_specs=[
            pl.BlockSpec((blk, n), col),
            pl.BlockSpec((blk, 1), col),
            scal,
            pl.BlockSpec((blk, 1), col),
            pl.BlockSpec((blk, 1), col),
            scal,
        ],
        out_specs=pl.BlockSpec((n, blk), lambda i: (0, i)),
        out_shape=jax.ShapeDtypeStruct((n, d), jnp.float32),
    )(agg, b, a, g, be, pa)


def _sc_scatter(y_t, edata):
    d, n = y_t.shape
    e = edata.shape[1]
    cols_per_tile = d // _NW
    pass_cols = 4
    npass = cols_per_tile // pass_cols
    chunk = 3200
    nchunk = e // chunk
    groups = chunk // _L

    mesh = plsc.VectorSubcoreMesh(core_axis_name="c", subcore_axis_name="s")

    @functools.partial(
        pl.kernel,
        out_type=jax.ShapeDtypeStruct((d, n), jnp.float32),
        mesh=mesh,
        compiler_params=pltpu.CompilerParams(needs_layout_passes=False,
                                             disable_bounds_checks=True),
        scratch_types=(
            [pltpu.VMEM((n,), jnp.float32)] * 4 +
            [pltpu.VMEM((n,), jnp.float32)] * 4 +
            [pltpu.VMEM((2, 3, chunk), jnp.int32),
             pltpu.SemaphoreType.DMA,
             pltpu.SemaphoreType.DMA]
        ),
    )
    def k(y_hbm, ed_hbm, out_hbm, xr0, xr1, xr2, xr3, ac0, ac1, ac2, ac3,
          ebuf, sem0, sem1):
        cid = lax.axis_index("c")
        sid = lax.axis_index("s")
        wid = sid * _NC + cid
        sems = (sem0, sem1)
        xrs = (xr0, xr1, xr2, xr3)
        acs = (ac0, ac1, ac2, ac3)

        def start(ch, p):
            off = pl.multiple_of(ch * chunk, 8)
            pltpu.async_copy(ed_hbm.at[:, pl.ds(off, chunk)], ebuf.at[p],
                             sems[p])

        def drain(p):
            pltpu.make_async_copy(ed_hbm.at[:, pl.ds(0, chunk)], ebuf.at[p],
                                  sems[p]).wait()

        gpi = 4

        def compute(p):
            def g_body(gi, c2):
                vals = []
                for g in range(gpi):
                    off = (gi * gpi + g) * _L
                    s_idx = ebuf[p, 0, pl.ds(off, _L)]
                    d_idx = ebuf[p, 1, pl.ds(off, _L)]
                    w = plsc.bitcast(ebuf[p, 2, pl.ds(off, _L)], jnp.float32)
                    gs = [plsc.load_gather(xrs[c], [s_idx])
                          for c in range(pass_cols)]
                    vals.append((d_idx, w, gs))
                for d_idx, w, gs in vals:
                    for c in range(pass_cols):
                        plsc.addupdate_scatter(acs[c], [d_idx], gs[c] * w)
                return c2

            lax.fori_loop(0, groups // gpi, g_body, 0)

        for p in range(npass):
            base = wid * cols_per_tile + p * pass_cols
            for c in range(pass_cols):
                pltpu.sync_copy(y_hbm.at[base + c], xrs[c])

            def zero_body(i, carry):
                zv = jnp.zeros((_L,), jnp.float32)
                for c in range(pass_cols):
                    acs[c][pl.ds(i * _L, _L)] = zv
                return carry

            lax.fori_loop(0, n // _L, zero_body, 0, unroll=5)

            start(0, 0)

            def pair_body(i2, carry):
                ch = i2 * 2
                start(ch + 1, 1)
                drain(0)
                compute(0)

                @pl.when(ch + 2 < nchunk)
                def _():
                    start(ch + 2, 0)

                drain(1)
                compute(1)
                return carry

            lax.fori_loop(0, nchunk // 2, pair_body, 0)
            for c in range(pass_cols):
                pltpu.sync_copy(acs[c], out_hbm.at[base + c])

    return k(y_t, edata)


def kernel(graph_edge_index, diff_edge_index, heat, edge_weight,
           W0, b0, a0, gamma0, beta0, pa0,
           W1, b1, a1, gamma1, beta1, pa1):
    ew_i32 = lax.bitcast_convert_type(edge_weight, jnp.int32)
    edata = jnp.concatenate(
        [diff_edge_index, ew_i32[None, :]], axis=0)
    d = W0.shape[1]

    b0c = b0.reshape(d, 1)
    g0c = gamma0.reshape(d, 1)
    be0c = beta0.reshape(d, 1)
    b1c = b1.reshape(d, 1)
    g1c = gamma1.reshape(d, 1)
    be1c = beta1.reshape(d, 1)
    a0s = a0.reshape(1, 1)
    pa0s = pa0.reshape(1, 1)
    a1s = a1.reshape(1, 1)
    pa1s = pa1.reshape(1, 1)

    y0 = _mm_in(W0.T, heat)
    agg0 = _sc_scatter(y0, edata)
    y1 = _mid(agg0, b0c, a0s, g0c, be0c, pa0s, W1)
    agg1 = _sc_scatter(y1, edata)
    return _final(agg1, b1c, a1s, g1c, be1c, pa1s)


# speedup vs baseline: 4.0438x; 1.0099x over previous
import functools

import jax
import jax.numpy as jnp
from jax import lax
from jax.experimental import pallas as pl
from jax.experimental.pallas import tpu as pltpu
from jax.experimental.pallas import tpu_sc as plsc

_NC = 2
_NS = 16
_NW = _NC * _NS
_L = 16

_EPS = 1e-5


def _mm_in_body(wt_ref, x_ref, o_ref):
    o_ref[...] = lax.dot_general(
        wt_ref[...], x_ref[...], (((1,), (1,)), ((), ())),
        preferred_element_type=jnp.float32, precision=lax.Precision.HIGHEST)


def _bn_chain(agg, b, a, g, be, pa):
    z = agg + b
    z = jnp.where(z > 0, z, a * z)
    n = z.shape[1]
    mu = jnp.sum(z, axis=1, keepdims=True) / n
    zc = z - mu
    var = jnp.sum(zc * zc, axis=1, keepdims=True) / n
    zn = g * zc * lax.rsqrt(var + _EPS) + be
    return jnp.where(zn > 0, zn, pa * zn)


def _mid_body(agg_ref, b_ref, a_ref, g_ref, be_ref, pa_ref, w_ref, o_ref):
    i = pl.program_id(0)
    zp = _bn_chain(agg_ref[...], b_ref[...], a_ref[0, 0], g_ref[...],
                   be_ref[...], pa_ref[0, 0])
    contrib = lax.dot_general(
        w_ref[...], zp, (((0,), (0,)), ((), ())),
        preferred_element_type=jnp.float32, precision=lax.Precision.HIGHEST)

    @pl.when(i == 0)
    def _():
        o_ref[...] = contrib

    @pl.when(i > 0)
    def _():
        o_ref[...] += contrib


def _final_body(agg_ref, b_ref, a_ref, g_ref, be_ref, pa_ref, o_ref):
    zp = _bn_chain(agg_ref[...], b_ref[...], a_ref[0, 0], g_ref[...],
                   be_ref[...], pa_ref[0, 0])
    o_ref[...] = zp.T


def _mm_in(w_t, x):
    dout, din = w_t.shape
    n = x.shape[0]
    blk = 64
    return pl.pallas_call(
        _mm_in_body,
        grid=(dout // blk,),
        in_specs=[
            pl.BlockSpec((blk, din), lambda i: (i, 0)),
            pl.BlockSpec((n, din), lambda i: (0, 0)),
        ],
        out_specs=pl.BlockSpec((blk, n), lambda i: (i, 0)),
        out_shape=jax.ShapeDtypeStruct((dout, n), jnp.float32),
    )(w_t, x)


def _mid(agg, b, a, g, be, pa, w):
    d, n = agg.shape
    dout = w.shape[1]
    blk = 64
    col = lambda i: (i, 0)
    scal = pl.BlockSpec((1, 1), lambda i: (0, 0), memory_space=pltpu.SMEM)
    return pl.pallas_call(
        _mid_body,
        grid=(d // blk,),
        in_specs=[
            pl.BlockSpec((blk, n), col),
            pl.BlockSpec((blk, 1), col),
            scal,
            pl.BlockSpec((blk, 1), col),
            pl.BlockSpec((blk, 1), col),
            scal,
            pl.BlockSpec((blk, dout), col),
        ],
        out_specs=pl.BlockSpec((dout, n), lambda i: (0, 0)),
        out_shape=jax.ShapeDtypeStruct((dout, n), jnp.float32),
    )(agg, b, a, g, be, pa, w)


def _final(agg, b, a, g, be, pa):
    d, n = agg.shape
    blk = 128
    col = lambda i: (i, 0)
    scal = pl.BlockSpec((1, 1), lambda i: (0, 0), memory_space=pltpu.SMEM)
    return pl.pallas_call(
        _final_body,
        grid=(d // blk,),
        in_specs=[
            pl.BlockSpec((blk, n), col),
            pl.BlockSpec((blk, 1), col),
            scal,
            pl.BlockSpec((blk, 1), col),
            pl.BlockSpec((blk, 1), col),
            scal,
        ],
        out_specs=pl.BlockSpec((n, blk), lambda i: (0, i)),
        out_shape=jax.ShapeDtypeStruct((n, d), jnp.float32),
    )(agg, b, a, g, be, pa)


def _sc_scatter(y_t, edata):
    d, n = y_t.shape
    e = edata.shape[1]
    cols_per_tile = d // _NW
    pass_cols = 4
    npass = cols_per_tile // pass_cols
    chunk = 3200
    nchunk = e // chunk
    groups = chunk // _L

    mesh = plsc.VectorSubcoreMesh(core_axis_name="c", subcore_axis_name="s")

    @functools.partial(
        pl.kernel,
        out_type=jax.ShapeDtypeStruct((d, n), jnp.float32),
        mesh=mesh,
        compiler_params=pltpu.CompilerParams(needs_layout_passes=False,
                                             disable_bounds_checks=True),
        scratch_types=(
            [pltpu.VMEM((n,), jnp.float32)] * 4 +
            [pltpu.VMEM((n,), jnp.float32)] * 4 +
            [pltpu.VMEM((2, 3, chunk), jnp.int32),
             pltpu.SemaphoreType.DMA,
             pltpu.SemaphoreType.DMA]
        ),
    )
    def k(y_hbm, ed_hbm, out_hbm, xr0, xr1, xr2, xr3, ac0, ac1, ac2, ac3,
          ebuf, sem0, sem1):
        cid = lax.axis_index("c")
        sid = lax.axis_index("s")
        wid = sid * _NC + cid
        sems = (sem0, sem1)
        xrs = (xr0, xr1, xr2, xr3)
        acs = (ac0, ac1, ac2, ac3)

        def start(ch, p):
            off = pl.multiple_of(ch * chunk, 8)
            pltpu.async_copy(ed_hbm.at[:, pl.ds(off, chunk)], ebuf.at[p],
                             sems[p])

        def drain(p):
            pltpu.make_async_copy(ed_hbm.at[:, pl.ds(0, chunk)], ebuf.at[p],
                                  sems[p]).wait()

        gpi = 4

        def compute(p):
            @plsc.parallel_loop(0, groups // gpi)
            def g_body(gi):
                vals = []
                for g in range(gpi):
                    off = (gi * gpi + g) * _L
                    s_idx = ebuf[p, 0, pl.ds(off, _L)]
                    d_idx = ebuf[p, 1, pl.ds(off, _L)]
                    w = plsc.bitcast(ebuf[p, 2, pl.ds(off, _L)], jnp.float32)
                    gs = [plsc.load_gather(xrs[c], [s_idx])
                          for c in range(pass_cols)]
                    vals.append((d_idx, w, gs))
                for d_idx, w, gs in vals:
                    for c in range(pass_cols):
                        plsc.addupdate_scatter(acs[c], [d_idx], gs[c] * w)

        for p in range(npass):
            base = wid * cols_per_tile + p * pass_cols
            for c in range(pass_cols):
                pltpu.sync_copy(y_hbm.at[base + c], xrs[c])

            def zero_body(i, carry):
                zv = jnp.zeros((_L,), jnp.float32)
                for c in range(pass_cols):
                    acs[c][pl.ds(i * _L, _L)] = zv
                return carry

            lax.fori_loop(0, n // _L, zero_body, 0, unroll=5)

            start(0, 0)

            def pair_body(i2, carry):
                ch = i2 * 2
                start(ch + 1, 1)
                drain(0)
                compute(0)

                @pl.when(ch + 2 < nchunk)
                def _():
                    start(ch + 2, 0)

                drain(1)
                compute(1)
                return carry

            lax.fori_loop(0, nchunk // 2, pair_body, 0)
            for c in range(pass_cols):
                pltpu.sync_copy(acs[c], out_hbm.at[base + c])

    return k(y_t, edata)


def kernel(graph_edge_index, diff_edge_index, heat, edge_weight,
           W0, b0, a0, gamma0, beta0, pa0,
           W1, b1, a1, gamma1, beta1, pa1):
    ew_i32 = lax.bitcast_convert_type(edge_weight, jnp.int32)
    edata = jnp.concatenate(
        [diff_edge_index, ew_i32[None, :]], axis=0)
    d = W0.shape[1]

    b0c = b0.reshape(d, 1)
    g0c = gamma0.reshape(d, 1)
    be0c = beta0.reshape(d, 1)
    b1c = b1.reshape(d, 1)
    g1c = gamma1.reshape(d, 1)
    be1c = beta1.reshape(d, 1)
    a0s = a0.reshape(1, 1)
    pa0s = pa0.reshape(1, 1)
    a1s = a1.reshape(1, 1)
    pa1s = pa1.reshape(1, 1)

    y0 = _mm_in(W0.T, heat)
    agg0 = _sc_scatter(y0, edata)
    y1 = _mid(agg0, b0c, a0s, g0c, be0c, pa0s, W1)
    agg1 = _sc_scatter(y1, edata)
    return _final(agg1, b1c, a1s, g1c, be1c, pa1s)


# speedup vs baseline: 4.7134x; 1.1656x over previous
import functools

import jax
import jax.numpy as jnp
from jax import lax
from jax.experimental import pallas as pl
from jax.experimental.pallas import tpu as pltpu
from jax.experimental.pallas import tpu_sc as plsc

_NC = 2
_NS = 16
_NW = _NC * _NS
_L = 16

_EPS = 1e-5


def _pack_pairs(top, bot):
    ue = lax.bitcast_convert_type(top.astype(jnp.bfloat16),
                                  jnp.uint16).astype(jnp.uint32)
    uo = lax.bitcast_convert_type(bot.astype(jnp.bfloat16),
                                  jnp.uint16).astype(jnp.uint32)
    return lax.bitcast_convert_type(ue | (uo << 16), jnp.int32)


def _mm_in_body(wta_ref, wtb_ref, x_ref, o_ref):
    dims = (((1,), (1,)), ((), ()))
    ya = lax.dot_general(wta_ref[...], x_ref[...], dims,
                         preferred_element_type=jnp.float32,
                         precision=lax.Precision.HIGHEST)
    yb = lax.dot_general(wtb_ref[...], x_ref[...], dims,
                         preferred_element_type=jnp.float32,
                         precision=lax.Precision.HIGHEST)
    o_ref[...] = _pack_pairs(ya, yb)


def _bn_chain(agg, b, a, g, be, pa):
    z = agg + b
    z = jnp.where(z > 0, z, a * z)
    n = z.shape[1]
    mu = jnp.sum(z, axis=1, keepdims=True) / n
    zc = z - mu
    var = jnp.sum(zc * zc, axis=1, keepdims=True) / n
    zn = g * zc * lax.rsqrt(var + _EPS) + be
    return jnp.where(zn > 0, zn, pa * zn)


def _mid_body(agg_ref, b_ref, a_ref, g_ref, be_ref, pa_ref, w_ref, o_ref,
              scr_ref):
    i = pl.program_id(0)
    nsteps = pl.num_programs(0)
    zp = _bn_chain(agg_ref[...], b_ref[...], a_ref[0, 0], g_ref[...],
                   be_ref[...], pa_ref[0, 0])
    contrib = lax.dot_general(
        w_ref[...], zp, (((0,), (0,)), ((), ())),
        preferred_element_type=jnp.float32, precision=lax.Precision.HIGHEST)

    @pl.when(i == 0)
    def _():
        scr_ref[...] = contrib

    @pl.when(i > 0)
    def _():
        scr_ref[...] += contrib

    @pl.when(i == nsteps - 1)
    def _():
        y = scr_ref[...]
        half = y.shape[0] // 2
        o_ref[...] = _pack_pairs(y[:half], y[half:])


def _final_body(agg_ref, b_ref, a_ref, g_ref, be_ref, pa_ref, o_ref):
    zp = _bn_chain(agg_ref[...], b_ref[...], a_ref[0, 0], g_ref[...],
                   be_ref[...], pa_ref[0, 0])
    o_ref[...] = zp.T


def _mm_in(w_t, x):
    dout, din = w_t.shape
    n = x.shape[0]
    half = dout // 2
    blk = 32
    col = lambda i: (i, 0)
    return pl.pallas_call(
        _mm_in_body,
        grid=(half // blk,),
        in_specs=[
            pl.BlockSpec((blk, din), col),
            pl.BlockSpec((blk, din), col),
            pl.BlockSpec((n, din), lambda i: (0, 0)),
        ],
        out_specs=pl.BlockSpec((blk, n), col),
        out_shape=jax.ShapeDtypeStruct((half, n), jnp.int32),
    )(w_t[:half], w_t[half:], x)


def _mid(agg, b, a, g, be, pa, w):
    d, n = agg.shape
    dout = w.shape[1]
    blk = 64
    col = lambda i: (i, 0)
    scal = pl.BlockSpec((1, 1), lambda i: (0, 0), memory_space=pltpu.SMEM)
    return pl.pallas_call(
        _mid_body,
        grid=(d // blk,),
        in_specs=[
            pl.BlockSpec((blk, n), col),
            pl.BlockSpec((blk, 1), col),
            scal,
            pl.BlockSpec((blk, 1), col),
            pl.BlockSpec((blk, 1), col),
            scal,
            pl.BlockSpec((blk, dout), col),
        ],
        out_specs=pl.BlockSpec((dout // 2, n), lambda i: (0, 0)),
        out_shape=jax.ShapeDtypeStruct((dout // 2, n), jnp.int32),
        scratch_shapes=[pltpu.VMEM((dout, n), jnp.float32)],
    )(agg, b, a, g, be, pa, w)


def _final(agg, b, a, g, be, pa):
    d, n = agg.shape
    blk = 128
    col = lambda i: (i, 0)
    scal = pl.BlockSpec((1, 1), lambda i: (0, 0), memory_space=pltpu.SMEM)
    return pl.pallas_call(
        _final_body,
        grid=(d // blk,),
        in_specs=[
            pl.BlockSpec((blk, n), col),
            pl.BlockSpec((blk, 1), col),
            scal,
            pl.BlockSpec((blk, 1), col),
            pl.BlockSpec((blk, 1), col),
            scal,
        ],
        out_specs=pl.BlockSpec((n, blk), lambda i: (0, i)),
        out_shape=jax.ShapeDtypeStruct((n, d), jnp.float32),
    )(agg, b, a, g, be, pa)


def _sc_scatter(y_pk, edata, d):
    dh, n = y_pk.shape
    e = edata.shape[1]
    prows = dh // _NW
    chunk = 640
    nchunk = e // chunk
    groups = chunk // _L
    gpi = 4
    assert nchunk % 2 == 0 and nchunk * chunk == e and groups % gpi == 0

    mesh = plsc.VectorSubcoreMesh(core_axis_name="c", subcore_axis_name="s")

    @functools.partial(
        pl.kernel,
        out_type=jax.ShapeDtypeStruct((d, n), jnp.float32),
        mesh=mesh,
        compiler_params=pltpu.CompilerParams(needs_layout_passes=False,
                                             disable_bounds_checks=True),
        scratch_types=(
            [pltpu.VMEM((n,), jnp.int32)] * prows +
            [pltpu.VMEM((n,), jnp.float32)] * (2 * prows) +
            [pltpu.VMEM((2, 2, chunk), jnp.int32),
             pltpu.SemaphoreType.DMA,
             pltpu.SemaphoreType.DMA]
        ),
    )
    def k(y_hbm, ed_hbm, out_hbm, xp0, xp1, xp2, xp3,
          ac0, ac1, ac2, ac3, ac4, ac5, ac6, ac7, ebuf, sem0, sem1):
        cid = lax.axis_index("c")
        sid = lax.axis_index("s")
        wid = sid * _NC + cid
        sems = (sem0, sem1)
        xps = (xp0, xp1, xp2, xp3)
        acs = (ac0, ac1, ac2, ac3, ac4, ac5, ac6, ac7)
        base = wid * prows

        def start(ch, p):
            off = pl.multiple_of(ch * chunk, 8)
            pltpu.async_copy(ed_hbm.at[:, pl.ds(off, chunk)], ebuf.at[p],
                             sems[p])

        def drain(p):
            pltpu.make_async_copy(ed_hbm.at[:, pl.ds(0, chunk)], ebuf.at[p],
                                  sems[p]).wait()

        def compute(p):
            @plsc.parallel_loop(0, groups // gpi)
            def g_body(gi):
                vals = []
                for g in range(gpi):
                    off = (gi * gpi + g) * _L
                    sd = ebuf[p, 0, pl.ds(off, _L)]
                    w = plsc.bitcast(ebuf[p, 1, pl.ds(off, _L)], jnp.float32)
                    s_idx = sd & 0xFFFF
                    d_idx = lax.shift_right_logical(sd, 16)
                    gs = [plsc.load_gather(xps[c], [s_idx])
                          for c in range(prows)]
                    vals.append((d_idx, w, gs))
                for d_idx, w, gs in vals:
                    for c in range(prows):
                        v_top = plsc.bitcast(gs[c] << 16, jnp.float32)
                        v_bot = plsc.bitcast(gs[c] & jnp.int32(-65536),
                                             jnp.float32)
                        plsc.addupdate_scatter(acs[c], [d_idx], v_top * w)
                        plsc.addupdate_scatter(acs[c + prows], [d_idx],
                                               v_bot * w)

        for c in range(prows):
            pltpu.sync_copy(y_hbm.at[base + c], xps[c])

        def zero_body(i, carry):
            zv = jnp.zeros((_L,), jnp.float32)
            for c in range(2 * prows):
                acs[c][pl.ds(i * _L, _L)] = zv
            return carry

        lax.fori_loop(0, n // _L, zero_body, 0, unroll=5)

        start(0, 0)

        def pair_body(i2, carry):
            ch = i2 * 2
            start(ch + 1, 1)
            drain(0)
            compute(0)

            @pl.when(ch + 2 < nchunk)
            def _():
                start(ch + 2, 0)

            drain(1)
            compute(1)
            return carry

        lax.fori_loop(0, nchunk // 2, pair_body, 0)
        for c in range(prows):
            pltpu.sync_copy(acs[c], out_hbm.at[base + c])
            pltpu.sync_copy(acs[c + prows], out_hbm.at[base + c + dh])

    return k(y_pk, edata)


def kernel(graph_edge_index, diff_edge_index, heat, edge_weight,
           W0, b0, a0, gamma0, beta0, pa0,
           W1, b1, a1, gamma1, beta1, pa1):
    src = diff_edge_index[0]
    dst = diff_edge_index[1]
    sd = src | (dst << 16)
    ew_i32 = lax.bitcast_convert_type(edge_weight, jnp.int32)
    edata = jnp.stack([sd, ew_i32])
    d = W0.shape[1]

    b0c = b0.reshape(d, 1)
    g0c = gamma0.reshape(d, 1)
    be0c = beta0.reshape(d, 1)
    b1c = b1.reshape(d, 1)
    g1c = gamma1.reshape(d, 1)
    be1c = beta1.reshape(d, 1)
    a0s = a0.reshape(1, 1)
    pa0s = pa0.reshape(1, 1)
    a1s = a1.reshape(1, 1)
    pa1s = pa1.reshape(1, 1)

    y0 = _mm_in(W0.T, heat)
    agg0 = _sc_scatter(y0, edata, d)
    y1 = _mid(agg0, b0c, a0s, g0c, be0c, pa0s, W1)
    agg1 = _sc_scatter(y1, edata, d)
    return _final(agg1, b1c, a1s, g1c, be1c, pa1s)


# speedup vs baseline: 4.8576x; 1.0306x over previous
import functools

import jax
import jax.numpy as jnp
from jax import lax
from jax.experimental import pallas as pl
from jax.experimental.pallas import tpu as pltpu
from jax.experimental.pallas import tpu_sc as plsc

_NC = 2
_NS = 16
_NW = _NC * _NS
_L = 16

_EPS = 1e-5


def _pack_pairs(top, bot):
    ue = lax.bitcast_convert_type(top.astype(jnp.bfloat16),
                                  jnp.uint16).astype(jnp.uint32)
    uo = lax.bitcast_convert_type(bot.astype(jnp.bfloat16),
                                  jnp.uint16).astype(jnp.uint32)
    return lax.bitcast_convert_type(ue | (uo << 16), jnp.int32)


def _mm_in_body(wta_ref, wtb_ref, x_ref, o_ref):
    dims = (((1,), (1,)), ((), ()))
    ya = lax.dot_general(wta_ref[...], x_ref[...], dims,
                         preferred_element_type=jnp.float32,
                         precision=lax.Precision.HIGHEST)
    yb = lax.dot_general(wtb_ref[...], x_ref[...], dims,
                         preferred_element_type=jnp.float32,
                         precision=lax.Precision.HIGHEST)
    o_ref[...] = _pack_pairs(ya, yb)


def _bn_chain(agg, b, a, g, be, pa):
    z = agg + b
    z = jnp.where(z > 0, z, a * z)
    n = z.shape[1]
    mu = jnp.sum(z, axis=1, keepdims=True) / n
    zc = z - mu
    var = jnp.sum(zc * zc, axis=1, keepdims=True) / n
    zn = g * zc * lax.rsqrt(var + _EPS) + be
    return jnp.where(zn > 0, zn, pa * zn)


def _mid_body(agg_ref, b_ref, a_ref, g_ref, be_ref, pa_ref, w_ref, o_ref,
              scr_ref):
    i = pl.program_id(0)
    nsteps = pl.num_programs(0)
    zp = _bn_chain(agg_ref[...], b_ref[...], a_ref[0, 0], g_ref[...],
                   be_ref[...], pa_ref[0, 0])
    contrib = lax.dot_general(
        w_ref[...], zp, (((0,), (0,)), ((), ())),
        preferred_element_type=jnp.float32, precision=lax.Precision.HIGHEST)

    @pl.when(i == 0)
    def _():
        scr_ref[...] = contrib

    @pl.when(i > 0)
    def _():
        scr_ref[...] += contrib

    @pl.when(i == nsteps - 1)
    def _():
        y = scr_ref[...]
        half = y.shape[0] // 2
        o_ref[...] = _pack_pairs(y[:half], y[half:])


def _final_body(agg_ref, b_ref, a_ref, g_ref, be_ref, pa_ref, o_ref):
    zp = _bn_chain(agg_ref[...], b_ref[...], a_ref[0, 0], g_ref[...],
                   be_ref[...], pa_ref[0, 0])
    o_ref[...] = zp.T


def _mm_in(w_t, x):
    dout, din = w_t.shape
    n = x.shape[0]
    half = dout // 2
    blk = 32
    col = lambda i: (i, 0)
    return pl.pallas_call(
        _mm_in_body,
        grid=(half // blk,),
        in_specs=[
            pl.BlockSpec((blk, din), col),
            pl.BlockSpec((blk, din), col),
            pl.BlockSpec((n, din), lambda i: (0, 0)),
        ],
        out_specs=pl.BlockSpec((blk, n), col),
        out_shape=jax.ShapeDtypeStruct((half, n), jnp.int32),
    )(w_t[:half], w_t[half:], x)


def _mid(agg, b, a, g, be, pa, w):
    d, n = agg.shape
    dout = w.shape[1]
    blk = 64
    col = lambda i: (i, 0)
    scal = pl.BlockSpec((1, 1), lambda i: (0, 0), memory_space=pltpu.SMEM)
    return pl.pallas_call(
        _mid_body,
        grid=(d // blk,),
        in_specs=[
            pl.BlockSpec((blk, n), col),
            pl.BlockSpec((blk, 1), col),
            scal,
            pl.BlockSpec((blk, 1), col),
            pl.BlockSpec((blk, 1), col),
            scal,
            pl.BlockSpec((blk, dout), col),
        ],
        out_specs=pl.BlockSpec((dout // 2, n), lambda i: (0, 0)),
        out_shape=jax.ShapeDtypeStruct((dout // 2, n), jnp.int32),
        scratch_shapes=[pltpu.VMEM((dout, n), jnp.float32)],
    )(agg, b, a, g, be, pa, w)


def _final(agg, b, a, g, be, pa):
    d, n = agg.shape
    blk = 128
    col = lambda i: (i, 0)
    scal = pl.BlockSpec((1, 1), lambda i: (0, 0), memory_space=pltpu.SMEM)
    return pl.pallas_call(
        _final_body,
        grid=(d // blk,),
        in_specs=[
            pl.BlockSpec((blk, n), col),
            pl.BlockSpec((blk, 1), col),
            scal,
            pl.BlockSpec((blk, 1), col),
            pl.BlockSpec((blk, 1), col),
            scal,
        ],
        out_specs=pl.BlockSpec((n, blk), lambda i: (0, i)),
        out_shape=jax.ShapeDtypeStruct((n, d), jnp.float32),
    )(agg, b, a, g, be, pa)


def _sc_scatter(y_pk, edata, d):
    dh, n = y_pk.shape
    e = edata.shape[1]
    prows = dh // _NW
    chunk = 1280
    nchunk = e // chunk
    groups = chunk // _L
    gpi = 4
    assert nchunk * chunk == e and groups % gpi == 0 and nchunk >= 2

    mesh = plsc.VectorSubcoreMesh(core_axis_name="c", subcore_axis_name="s")

    @functools.partial(
        pl.kernel,
        out_type=jax.ShapeDtypeStruct((d, n), jnp.float32),
        mesh=mesh,
        compiler_params=pltpu.CompilerParams(needs_layout_passes=False,
                                             disable_bounds_checks=True),
        scratch_types=(
            [pltpu.VMEM((n,), jnp.int32)] * prows +
            [pltpu.VMEM((n,), jnp.float32)] * (2 * prows) +
            [pltpu.VMEM((2, 2, chunk), jnp.int32),
             pltpu.SemaphoreType.DMA,
             pltpu.SemaphoreType.DMA]
        ),
    )
    def k(y_hbm, ed_hbm, out_hbm, xp0, xp1, xp2, xp3,
          ac0, ac1, ac2, ac3, ac4, ac5, ac6, ac7, ebuf, sem0, sem1):
        cid = lax.axis_index("c")
        sid = lax.axis_index("s")
        wid = sid * _NC + cid
        sems = (sem0, sem1)
        xps = (xp0, xp1, xp2, xp3)
        acs = (ac0, ac1, ac2, ac3, ac4, ac5, ac6, ac7)
        base = wid * prows

        def start(ch, p):
            off = pl.multiple_of(ch * chunk, 8)
            pltpu.async_copy(ed_hbm.at[:, pl.ds(off, chunk)], ebuf.at[p],
                             sems[p])

        def drain(p):
            pltpu.make_async_copy(ed_hbm.at[:, pl.ds(0, chunk)], ebuf.at[p],
                                  sems[p]).wait()

        def compute(p):
            @plsc.parallel_loop(0, groups // gpi)
            def g_body(gi):
                vals = []
                for g in range(gpi):
                    off = (gi * gpi + g) * _L
                    sd = ebuf[p, 0, pl.ds(off, _L)]
                    w = plsc.bitcast(ebuf[p, 1, pl.ds(off, _L)], jnp.float32)
                    s_idx = sd & 0xFFFF
                    d_idx = lax.shift_right_logical(sd, 16)
                    gs = [plsc.load_gather(xps[c], [s_idx])
                          for c in range(prows)]
                    vals.append((d_idx, w, gs))
                for d_idx, w, gs in vals:
                    for c in range(prows):
                        v_top = plsc.bitcast(gs[c] << 16, jnp.float32)
                        v_bot = plsc.bitcast(gs[c] & jnp.int32(-65536),
                                             jnp.float32)
                        plsc.addupdate_scatter(acs[c], [d_idx], v_top * w)
                        plsc.addupdate_scatter(acs[c + prows], [d_idx],
                                               v_bot * w)

        for c in range(prows):
            pltpu.sync_copy(y_hbm.at[base + c], xps[c])

        def zero_body(i, carry):
            zv = jnp.zeros((_L,), jnp.float32)
            for c in range(2 * prows):
                acs[c][pl.ds(i * _L, _L)] = zv
            return carry

        lax.fori_loop(0, n // _L, zero_body, 0, unroll=5)

        start(0, 0)

        def pair_body(i2, carry):
            ch = i2 * 2
            start(ch + 1, 1)
            drain(0)
            compute(0)

            @pl.when(ch + 2 < nchunk)
            def _():
                start(ch + 2, 0)

            drain(1)
            compute(1)
            return carry

        lax.fori_loop(0, nchunk // 2, pair_body, 0)
        if nchunk % 2:
            drain(0)
            compute(0)
        for c in range(prows):
            pltpu.sync_copy(acs[c], out_hbm.at[base + c])
            pltpu.sync_copy(acs[c + prows], out_hbm.at[base + c + dh])

    return k(y_pk, edata)


def kernel(graph_edge_index, diff_edge_index, heat, edge_weight,
           W0, b0, a0, gamma0, beta0, pa0,
           W1, b1, a1, gamma1, beta1, pa1):
    src = diff_edge_index[0]
    dst = diff_edge_index[1]
    sd = src | (dst << 16)
    ew_i32 = lax.bitcast_convert_type(edge_weight, jnp.int32)
    edata = jnp.stack([sd, ew_i32])
    d = W0.shape[1]

    b0c = b0.reshape(d, 1)
    g0c = gamma0.reshape(d, 1)
    be0c = beta0.reshape(d, 1)
    b1c = b1.reshape(d, 1)
    g1c = gamma1.reshape(d, 1)
    be1c = beta1.reshape(d, 1)
    a0s = a0.reshape(1, 1)
    pa0s = pa0.reshape(1, 1)
    a1s = a1.reshape(1, 1)
    pa1s = pa1.reshape(1, 1)

    y0 = _mm_in(W0.T, heat)
    agg0 = _sc_scatter(y0, edata, d)
    y1 = _mid(agg0, b0c, a0s, g0c, be0c, pa0s, W1)
    agg1 = _sc_scatter(y1, edata, d)
    return _final(agg1, b1c, a1s, g1c, be1c, pa1s)
